# baseline (speedup 1.0000x reference)
import functools

import jax
import jax.numpy as jnp
from jax import lax
from jax.experimental import pallas as pl
from jax.experimental.pallas import tpu as pltpu
from jax.experimental.pallas import tpu_sc as plsc

_NUM_CORES = 2
_NUM_SUBCORES = 16
_NUM_WORKERS = _NUM_CORES * _NUM_SUBCORES

_V_TILE = 2048


@functools.lru_cache(maxsize=None)
def _make_gather(vocab, embed, batch):
    assert batch % (8 * _NUM_WORKERS) == 0
    b_per_w = batch // _NUM_WORKERS
    mesh = plsc.VectorSubcoreMesh(core_axis_name="c", subcore_axis_name="s")

    def body(table_hbm, idx_hbm, out_hbm, idx_v, rows_v, sem):
        wid = lax.axis_index("s") * _NUM_CORES + lax.axis_index("c")
        base = wid * b_per_w
        pltpu.sync_copy(idx_hbm.at[pl.ds(base, b_per_w)], idx_v)
        pltpu.async_copy(table_hbm.at[idx_v], rows_v, sem).wait()
        pltpu.sync_copy(rows_v, out_hbm.at[pl.ds(base, b_per_w)])

    return pl.kernel(
        body,
        out_type=jax.ShapeDtypeStruct((batch, embed), jnp.float32),
        mesh=mesh,
        scratch_types=[
            pltpu.VMEM((b_per_w,), jnp.int32),
            pltpu.VMEM((b_per_w, embed), jnp.float32),
            pltpu.SemaphoreType.DMA,
        ],
        compiler_params=pltpu.CompilerParams(use_tc_tiling_on_sc=False),
    )


def _proj_body(x_ref, w_ref, b_ref, out_ref):
    out_ref[...] = lax.dot_general(
        x_ref[...],
        w_ref[...],
        dimension_numbers=(((1,), (1,)), ((), ())),
        preferred_element_type=jnp.float32,
    ) + b_ref[...]


@functools.lru_cache(maxsize=None)
def _make_proj(batch, embed, vocab, interpret=False):
    grid = (pl.cdiv(vocab, _V_TILE),)
    return pl.pallas_call(
        _proj_body,
        grid=grid,
        in_specs=[
            pl.BlockSpec((batch, embed), lambda j: (0, 0)),
            pl.BlockSpec((_V_TILE, embed), lambda j: (j, 0)),
            pl.BlockSpec((1, _V_TILE), lambda j: (0, j)),
        ],
        out_specs=pl.BlockSpec((batch, _V_TILE), lambda j: (0, j)),
        out_shape=jax.ShapeDtypeStruct((batch, vocab), jnp.float32),
        interpret=interpret,
    )


def kernel(input_ids, token_embedding, head_w, head_b):
    vocab, embed = token_embedding.shape
    (batch,) = input_ids.shape
    idx = input_ids.astype(jnp.int32)
    x = _make_gather(vocab, embed, batch)(token_embedding, idx)
    proj = _make_proj(batch, embed, vocab)
    return proj(x, head_w, head_b.reshape(1, vocab))


# speedup vs baseline: 2.2237x; 2.2237x over previous
import functools

import jax
import jax.numpy as jnp
from jax import lax
from jax.experimental import pallas as pl
from jax.experimental.pallas import tpu as pltpu
from jax.experimental.pallas import tpu_sc as plsc

_NUM_CORES = 2
_NUM_SUBCORES = 16
_NUM_WORKERS = _NUM_CORES * _NUM_SUBCORES

_V_TILE = 2048


@functools.lru_cache(maxsize=None)
def _make_gather(vocab, embed, batch):
    assert batch % (8 * _NUM_WORKERS) == 0
    b_per_w = batch // _NUM_WORKERS
    mesh = plsc.VectorSubcoreMesh(core_axis_name="c", subcore_axis_name="s")

    def body(table_hbm, idx_hbm, out_hbm, idx_v, rows_v, sem):
        wid = lax.axis_index("s") * _NUM_CORES + lax.axis_index("c")
        base = wid * b_per_w
        pltpu.sync_copy(idx_hbm.at[pl.ds(base, b_per_w)], idx_v)
        pltpu.async_copy(table_hbm.at[idx_v], rows_v, sem).wait()
        pltpu.sync_copy(rows_v, out_hbm.at[pl.ds(base, b_per_w)])

    return pl.kernel(
        body,
        out_type=jax.ShapeDtypeStruct((batch, embed), jnp.float32),
        mesh=mesh,
        scratch_types=[
            pltpu.VMEM((b_per_w,), jnp.int32),
            pltpu.VMEM((b_per_w, embed), jnp.float32),
            pltpu.SemaphoreType.DMA,
        ],
        compiler_params=pltpu.CompilerParams(use_tc_tiling_on_sc=False),
    )


def _proj_body(wt_ref, xt_ref, b_ref, out_ref):
    out_ref[...] = lax.dot_general(
        wt_ref[...],
        xt_ref[...],
        dimension_numbers=(((0,), (0,)), ((), ())),
        preferred_element_type=jnp.float32,
    ) + b_ref[...]


@functools.lru_cache(maxsize=None)
def _make_proj(batch, embed, vocab, interpret=False):
    grid = (pl.cdiv(vocab, _V_TILE),)
    return pl.pallas_call(
        _proj_body,
        grid=grid,
        in_specs=[
            pl.BlockSpec((embed, _V_TILE), lambda j: (0, j)),
            pl.BlockSpec((embed, batch), lambda j: (0, 0)),
            pl.BlockSpec((_V_TILE, 1), lambda j: (j, 0)),
        ],
        out_specs=pl.BlockSpec((_V_TILE, batch), lambda j: (j, 0)),
        out_shape=jax.ShapeDtypeStruct((vocab, batch), jnp.float32),
        interpret=interpret,
    )


def kernel(input_ids, token_embedding, head_w, head_b):
    vocab, embed = token_embedding.shape
    (batch,) = input_ids.shape
    idx = input_ids.astype(jnp.int32)
    x = _make_gather(vocab, embed, batch)(token_embedding, idx)
    proj = _make_proj(batch, embed, vocab)
    out_t = proj(
        head_w.T,
        x.T,
        head_b.reshape(vocab, 1),
    )
    return out_t.T


# speedup vs baseline: 2.8220x; 1.2691x over previous
import functools

import jax
import jax.numpy as jnp
from jax import lax
from jax.experimental import pallas as pl
from jax.experimental.pallas import tpu as pltpu
from jax.experimental.pallas import tpu_sc as plsc

_NUM_CORES = 2
_NUM_SUBCORES = 16
_NUM_WORKERS = _NUM_CORES * _NUM_SUBCORES

_V_TILE = 2048


@functools.lru_cache(maxsize=None)
def _make_gather(vocab, embed, batch):
    assert batch % (8 * _NUM_WORKERS) == 0
    b_per_w = batch // _NUM_WORKERS
    mesh = plsc.VectorSubcoreMesh(core_axis_name="c", subcore_axis_name="s")

    def body(table_hbm, idx_hbm, out_hbm, idx_v, rows_v, sem):
        wid = lax.axis_index("s") * _NUM_CORES + lax.axis_index("c")
        base = wid * b_per_w
        pltpu.sync_copy(idx_hbm.at[pl.ds(base, b_per_w)], idx_v)
        pltpu.async_copy(table_hbm.at[idx_v], rows_v, sem).wait()
        pltpu.sync_copy(rows_v, out_hbm.at[pl.ds(base, b_per_w)])

    return pl.kernel(
        body,
        out_type=jax.ShapeDtypeStruct((batch, embed), jnp.float32),
        mesh=mesh,
        scratch_types=[
            pltpu.VMEM((b_per_w,), jnp.int32),
            pltpu.VMEM((b_per_w, embed), jnp.float32),
            pltpu.SemaphoreType.DMA,
        ],
        compiler_params=pltpu.CompilerParams(use_tc_tiling_on_sc=False),
    )


def _proj_body(wt_ref, xt_ref, b_ref, out_ref):
    acc = lax.dot_general(
        wt_ref[...],
        xt_ref[...],
        dimension_numbers=(((0,), (0,)), ((), ())),
        preferred_element_type=jnp.float32,
    )
    ones = jnp.ones((1, acc.shape[1]), jnp.float32)
    out_ref[...] = acc + lax.dot_general(
        b_ref[...],
        ones,
        dimension_numbers=(((0,), (0,)), ((), ())),
        preferred_element_type=jnp.float32,
    )


@functools.lru_cache(maxsize=None)
def _make_proj(batch, embed, vocab, interpret=False):
    grid = (pl.cdiv(vocab, _V_TILE),)
    return pl.pallas_call(
        _proj_body,
        grid=grid,
        in_specs=[
            pl.BlockSpec((embed, _V_TILE), lambda j: (0, j)),
            pl.BlockSpec((embed, batch), lambda j: (0, 0)),
            pl.BlockSpec((1, _V_TILE), lambda j: (0, j)),
        ],
        out_specs=pl.BlockSpec((_V_TILE, batch), lambda j: (j, 0)),
        out_shape=jax.ShapeDtypeStruct((vocab, batch), jnp.float32),
        interpret=interpret,
    )


def kernel(input_ids, token_embedding, head_w, head_b):
    vocab, embed = token_embedding.shape
    (batch,) = input_ids.shape
    idx = input_ids.astype(jnp.int32)
    x = _make_gather(vocab, embed, batch)(token_embedding, idx)
    proj = _make_proj(batch, embed, vocab)
    out_t = proj(
        head_w.T,
        x.T,
        head_b.reshape(1, vocab),
    )
    return out_t.T


# speedup vs baseline: 3.1816x; 1.1274x over previous
import functools

import jax
import jax.numpy as jnp
from jax import lax
from jax.experimental import pallas as pl
from jax.experimental.pallas import tpu as pltpu
from jax.experimental.pallas import tpu_sc as plsc

_NUM_CORES = 2
_NUM_SUBCORES = 16
_NUM_WORKERS = _NUM_CORES * _NUM_SUBCORES

_V_TILE = 2048


@functools.lru_cache(maxsize=None)
def _make_gather_t(vocab, embed, batch):
    assert embed % _NUM_WORKERS == 0
    k_per_w = embed // _NUM_WORKERS
    n_chunks = batch // 128
    assert batch % 128 == 0
    mesh = plsc.VectorSubcoreMesh(core_axis_name="c", subcore_axis_name="s")

    def body(table_hbm, idx_hbm, out_hbm, idx_v, fidx_v, row_v, sem):
        wid = lax.axis_index("s") * _NUM_CORES + lax.axis_index("c")
        pltpu.sync_copy(idx_hbm, idx_v)
        for kk in range(k_per_w):
            k = wid * k_per_w + kk
            koff = jnp.int32(k * vocab)
            for i in range(batch // 16):
                fidx_v[pl.ds(i * 16, 16)] = idx_v[pl.ds(i * 16, 16)] + koff
            for c in range(n_chunks):
                pltpu.async_copy(
                    table_hbm.at[fidx_v.at[pl.ds(c * 128, 128)]],
                    row_v.at[pl.ds(c * 128, 128)],
                    sem,
                )
            for c in range(n_chunks):
                pltpu.make_async_copy(
                    table_hbm.at[fidx_v.at[pl.ds(c * 128, 128)]],
                    row_v.at[pl.ds(c * 128, 128)],
                    sem,
                ).wait()
            pltpu.sync_copy(row_v, out_hbm.at[k])

    return pl.kernel(
        body,
        out_type=jax.ShapeDtypeStruct((embed, batch), jnp.float32),
        mesh=mesh,
        scratch_types=[
            pltpu.VMEM((batch,), jnp.int32),
            pltpu.VMEM((batch,), jnp.int32),
            pltpu.VMEM((batch,), jnp.float32),
            pltpu.SemaphoreType.DMA,
        ],
        compiler_params=pltpu.CompilerParams(use_tc_tiling_on_sc=False),
    )


def _proj_body(wt_ref, xt_ref, b_ref, out_ref):
    acc = lax.dot_general(
        wt_ref[...],
        xt_ref[...],
        dimension_numbers=(((0,), (0,)), ((), ())),
        preferred_element_type=jnp.float32,
    )
    ones = jnp.ones((1, acc.shape[1]), jnp.float32)
    out_ref[...] = acc + lax.dot_general(
        b_ref[...],
        ones,
        dimension_numbers=(((0,), (0,)), ((), ())),
        preferred_element_type=jnp.float32,
    )


@functools.lru_cache(maxsize=None)
def _make_proj(batch, embed, vocab, interpret=False):
    grid = (pl.cdiv(vocab, _V_TILE),)
    return pl.pallas_call(
        _proj_body,
        grid=grid,
        in_specs=[
            pl.BlockSpec((embed, _V_TILE), lambda j: (0, j)),
            pl.BlockSpec((embed, batch), lambda j: (0, 0)),
            pl.BlockSpec((1, _V_TILE), lambda j: (0, j)),
        ],
        out_specs=pl.BlockSpec((_V_TILE, batch), lambda j: (j, 0)),
        out_shape=jax.ShapeDtypeStruct((vocab, batch), jnp.float32),
        interpret=interpret,
    )


def kernel(input_ids, token_embedding, head_w, head_b):
    vocab, embed = token_embedding.shape
    (batch,) = input_ids.shape
    idx = input_ids.astype(jnp.int32)
    table_t_flat = token_embedding.T.reshape(vocab * embed)
    xt = _make_gather_t(vocab, embed, batch)(table_t_flat, idx)
    proj = _make_proj(batch, embed, vocab)
    out_t = proj(
        head_w.T,
        xt,
        head_b.reshape(1, vocab),
    )
    return out_t.T
